# Initial kernel scaffold; baseline (speedup 1.0000x reference)
import jax
import jax.numpy as jnp
from jax.experimental import pallas as pl


def kernel(x):
    raise NotImplementedError("write your pallas kernel here")


# speedup vs baseline: 10.6790x; 10.6790x over previous
import jax
import jax.numpy as jnp
from jax.experimental import pallas as pl
from jax.experimental.pallas import tpu as pltpu

_K = 512
_ROWS_PER_BLOCK = 8
def _body(x_ref, o_ref):
    _SIGN = jnp.int32(-(2 ** 31))
    x = x_ref[...]
    bits = jax.lax.bitcast_convert_type(x, jnp.int32)
    key = bits ^ (jax.lax.shift_right_arithmetic(bits, 31) & jnp.int32(0x7FFFFFFF))

    nrows = x.shape[0]

    def step(i, t_u):
        bit = jnp.int32(1) << (jnp.int32(31) - i)
        cand_u = t_u | bit
        cand = cand_u ^ _SIGN
        cnt = jnp.sum((key >= cand).astype(jnp.int32), axis=1, keepdims=True)
        return jnp.where(cnt >= _K, cand_u, t_u)

    t_u = jax.lax.fori_loop(0, 32, step, jnp.zeros((nrows, 1), jnp.int32))
    thr = t_u ^ _SIGN
    thr = jnp.maximum(thr, jnp.int32(1))
    o_ref[...] = jnp.where(key >= thr, x, 0.0)


def kernel(x):
    n_rows, n_cols = x.shape
    grid = (n_rows // _ROWS_PER_BLOCK,)
    return pl.pallas_call(
        _body,
        grid=grid,
        in_specs=[pl.BlockSpec((_ROWS_PER_BLOCK, n_cols), lambda i: (i, 0))],
        out_specs=pl.BlockSpec((_ROWS_PER_BLOCK, n_cols), lambda i: (i, 0)),
        out_shape=jax.ShapeDtypeStruct((n_rows, n_cols), x.dtype),
    )(x)
